# Initial kernel scaffold; baseline (speedup 1.0000x reference)
import jax
import jax.numpy as jnp
from jax.experimental import pallas as pl


def kernel(x, segment_input_ids, segment_table, position_table):
    raise NotImplementedError("write your pallas kernel here")


# speedup vs baseline: 3.0045x; 3.0045x over previous
import jax
import jax.numpy as jnp
from jax.experimental import pallas as pl

_B, _S, _D = 4, 2048, 1024
_TS = 256


def _body(x_ref, m_ref, seg_ref, pos_ref, o_ref):
    s0 = seg_ref[0:1, :][None]
    s1 = seg_ref[1:2, :][None]
    m = m_ref[...]
    o_ref[...] = x_ref[...] + pos_ref[...][None] + s0 + m * (s1 - s0)


def kernel(x, segment_input_ids, segment_table, position_table):
    m = segment_input_ids.astype(jnp.float32)[..., None]
    return pl.pallas_call(
        _body,
        grid=(_S // _TS,),
        in_specs=[
            pl.BlockSpec((_B, _TS, _D), lambda i: (0, i, 0)),
            pl.BlockSpec((_B, _TS, 1), lambda i: (0, i, 0)),
            pl.BlockSpec((2, _D), lambda i: (0, 0)),
            pl.BlockSpec((_TS, _D), lambda i: (i, 0)),
        ],
        out_specs=pl.BlockSpec((_B, _TS, _D), lambda i: (0, i, 0)),
        out_shape=jax.ShapeDtypeStruct((_B, _S, _D), jnp.float32),
    )(x, m, segment_table, position_table)
